# Initial kernel scaffold; baseline (speedup 1.0000x reference)
import jax
import jax.numpy as jnp
from jax.experimental import pallas as pl


def kernel(D, edge_index, edge_weights):
    raise NotImplementedError("write your pallas kernel here")


# speedup vs baseline: 28.0756x; 28.0756x over previous
import functools

import jax
import jax.numpy as jnp
from jax import lax
from jax.experimental import pallas as pl
from jax.experimental.pallas import tpu as pltpu
from jax.experimental.pallas import tpu_sc as plsc

_N = 100000
_E = 3200000
_C = 8
_LR = 1e-3
_NC = 2
_NS = 16
_NW = _NC * _NS
_EW = _E // _NW
_CH = 400
_NCHUNK = _EW // _CH
_RC = 1000
_NRC = _N // _RC
_VPC = _CH * _C // 16


def _edge_scratch(with_gather, with_residual):
    scr = []
    if with_gather:
        scr.append(pltpu.VMEM_SHARED((_N, _C), jnp.float32))
    scr += [
        pltpu.VMEM_SHARED((_N, _C), jnp.float32),
        pltpu.VMEM((_CH,), jnp.int32),
        pltpu.VMEM((_CH,), jnp.int32),
        pltpu.VMEM((_CH,), jnp.float32),
        pltpu.VMEM((_CH, _C), jnp.float32),
        pltpu.VMEM((_CH, _C), jnp.float32),
    ]
    if with_gather:
        scr += [
            pltpu.VMEM((_CH, _C), jnp.float32),
            pltpu.VMEM((_CH, _C), jnp.float32),
        ]
    else:
        scr.append(pltpu.VMEM((_CH, _C), jnp.float32))
    if with_residual:
        scr += [
            pltpu.VMEM((_CH,), jnp.float32),
            pltpu.VMEM((_CH, _C), jnp.float32),
            pltpu.VMEM((_CH, _C), jnp.float32),
        ]
    scr.append(pltpu.SemaphoreType.DMA)
    return scr


def _make_edge_kernel(with_gather, with_residual):
    mesh = plsc.VectorSubcoreMesh(core_axis_name="c", subcore_axis_name="s")
    out_type = [jax.ShapeDtypeStruct((_NC, _N, _C), jnp.float32)]
    if with_residual:
        out_type.append(jax.ShapeDtypeStruct((_E, _C), jnp.float32))

    def body(*refs):
        it = iter(refs)
        main_hbm = next(it)
        src_hbm = next(it)
        dst_hbm = next(it)
        scale_hbm = next(it)
        if with_residual:
            w_hbm = next(it)
            d_hbm = next(it)
        zero_hbm = next(it)
        p_out = next(it)
        if with_residual:
            r_out = next(it)
        if with_gather:
            zloc = next(it)
        acc = next(it)
        srcb = next(it)
        dstb = next(it)
        scaleb = next(it)
        mb = next(it)
        nmb = next(it)
        if with_gather:
            zsb = next(it)
            zdb = next(it)
        else:
            db = next(it)
        if with_residual:
            wb = next(it)
            db = next(it)
            rb = next(it)
        sem = next(it)

        c = lax.axis_index("c")
        s = lax.axis_index("s")

        for j in range(-(-_NRC // _NS)):
            cid = s + _NS * j

            @pl.when(cid < _NRC)
            def _():
                r0 = cid * _RC
                if with_gather:
                    pltpu.sync_copy(
                        main_hbm.at[pl.ds(r0, _RC), :],
                        zloc.at[pl.ds(r0, _RC), :],
                    )
                pltpu.sync_copy(
                    zero_hbm.at[pl.ds(r0, _RC), :],
                    acc.at[pl.ds(r0, _RC), :],
                )

        plsc.subcore_barrier()

        base = (c * _NS + s) * _EW
        lanes = lax.iota(jnp.int32, 16)
        s_row = lanes >> 3
        s_col = lanes & 7

        def chunk_body(t, carry):
            off = base + t * _CH
            pltpu.sync_copy(src_hbm.at[pl.ds(off, _CH)], srcb)
            pltpu.sync_copy(dst_hbm.at[pl.ds(off, _CH)], dstb)
            pltpu.sync_copy(scale_hbm.at[pl.ds(off, _CH)], scaleb)
            if with_residual:
                pltpu.sync_copy(w_hbm.at[pl.ds(off, _CH)], wb)
            if with_gather:
                pltpu.async_copy(zloc.at[srcb], zsb, sem).wait()
                pltpu.async_copy(zloc.at[dstb], zdb, sem).wait()
            else:
                pltpu.sync_copy(main_hbm.at[pl.ds(off, _CH), :], db)
            if with_residual:
                pltpu.sync_copy(d_hbm.at[pl.ds(off, _CH), :], db)

            def vbody(v, carry2):
                rows = v * 2 + s_row
                a = plsc.load_gather(scaleb, [rows])
                if with_gather:
                    zs = plsc.load_gather(zsb, [rows, s_col])
                    zd = plsc.load_gather(zdb, [rows, s_col])
                    diff = zs - zd
                    m = a * diff
                    if with_residual:
                        wv = plsc.load_gather(wb, [rows])
                        dv = plsc.load_gather(db, [rows, s_col])
                        plsc.store_scatter(rb, [rows, s_col], wv * diff - dv)
                else:
                    dv = plsc.load_gather(db, [rows, s_col])
                    m = a * dv
                plsc.store_scatter(mb, [rows, s_col], m)
                plsc.store_scatter(nmb, [rows, s_col], -m)
                return carry2

            lax.fori_loop(0, _VPC, vbody, 0, unroll=8)

            pltpu.sync_copy(mb, acc.at[srcb], add=True)
            pltpu.sync_copy(nmb, acc.at[dstb], add=True)
            if with_residual:
                pltpu.sync_copy(rb, r_out.at[pl.ds(off, _CH), :])
            return carry

        lax.fori_loop(0, _NCHUNK, chunk_body, 0)
        plsc.subcore_barrier()

        for j in range(-(-_NRC // _NS)):
            cid = s + _NS * j

            @pl.when(cid < _NRC)
            def _():
                r0 = cid * _RC
                pltpu.sync_copy(
                    acc.at[pl.ds(r0, _RC), :],
                    p_out.at[c, pl.ds(r0, _RC), :],
                )

    return pl.kernel(
        body,
        out_type=out_type,
        mesh=mesh,
        scratch_types=_edge_scratch(with_gather, with_residual),
        compiler_params=pltpu.CompilerParams(
            needs_layout_passes=False, use_tc_tiling_on_sc=False
        ),
    )


def _merge(z, p, pb):
    zr = z.reshape(_N * _C // 128, 128)
    pr = p.reshape(_NC, _N * _C // 128, 128)
    pbr = pb.reshape(_NC, _N * _C // 128, 128)

    def body(z_ref, p_ref, pb_ref, o_ref):
        o_ref[...] = (
            z_ref[...] - p_ref[0] - p_ref[1] + pb_ref[0] + pb_ref[1]
        )

    out = pl.pallas_call(
        body,
        out_shape=jax.ShapeDtypeStruct((_N * _C // 128, 128), jnp.float32),
    )(zr, pr, pbr)
    return out.reshape(_N, _C)


def kernel(D, edge_index, edge_weights):
    src = edge_index[0]
    dst = edge_index[1]
    w = edge_weights
    aw = _LR * w
    aw2 = aw * w

    b_pass = _make_edge_kernel(with_gather=False, with_residual=False)
    l_pass = _make_edge_kernel(with_gather=True, with_residual=False)
    lr_pass = _make_edge_kernel(with_gather=True, with_residual=True)

    zeros_n = jnp.zeros((_N, _C), jnp.float32)
    zeros_p = jnp.zeros((_NC, _N, _C), jnp.float32)
    [pb] = b_pass(D, src, dst, aw, zeros_n)
    z = _merge(zeros_n, zeros_p, pb)
    for _ in range(8):
        [p] = l_pass(z, src, dst, aw2, zeros_n)
        z = _merge(z, p, pb)
    p, r = lr_pass(z, src, dst, aw2, w, D, zeros_n)
    z = _merge(z, p, pb)
    return z, zeros_n, r


# speedup vs baseline: 36.5814x; 1.3030x over previous
import functools

import jax
import jax.numpy as jnp
from jax import lax
from jax.experimental import pallas as pl
from jax.experimental.pallas import tpu as pltpu
from jax.experimental.pallas import tpu_sc as plsc

_N = 100000
_E = 3200000
_C = 8
_LR = 1e-3
_NC = 2
_NS = 16
_NW = _NC * _NS
_EW = _E // _NW
_CH = 400
_NCHUNK = _EW // _CH
_RC = 1000
_NRC = _N // _RC
_VPC = _CH * _C // 16


def _edge_scratch(with_gather, with_residual):
    scr = []
    if with_gather:
        scr.append(pltpu.VMEM_SHARED((_N, _C), jnp.float32))
    scr += [
        pltpu.VMEM_SHARED((_N, _C), jnp.float32),
        pltpu.VMEM((_CH,), jnp.int32),
        pltpu.VMEM((_CH,), jnp.int32),
        pltpu.VMEM((_CH,), jnp.float32),
        pltpu.VMEM((_CH, _C), jnp.float32),
        pltpu.VMEM((_CH, _C), jnp.float32),
    ]
    if with_gather:
        scr += [
            pltpu.VMEM((_CH, _C), jnp.float32),
            pltpu.VMEM((_CH, _C), jnp.float32),
        ]
    else:
        scr.append(pltpu.VMEM((_CH, _C), jnp.float32))
    if with_residual:
        scr += [
            pltpu.VMEM((_CH,), jnp.float32),
            pltpu.VMEM((_CH, _C), jnp.float32),
            pltpu.VMEM((_CH, _C), jnp.float32),
        ]
    scr.append(pltpu.SemaphoreType.DMA)
    scr.append(pltpu.SemaphoreType.DMA)
    scr.append(pltpu.SemaphoreType.DMA)
    return scr


def _make_edge_kernel(with_gather, with_residual):
    mesh = plsc.VectorSubcoreMesh(core_axis_name="c", subcore_axis_name="s")
    out_type = [jax.ShapeDtypeStruct((_NC, _N, _C), jnp.float32)]
    if with_residual:
        out_type.append(jax.ShapeDtypeStruct((_E, _C), jnp.float32))

    def body(*refs):
        it = iter(refs)
        main_hbm = next(it)
        src_hbm = next(it)
        dst_hbm = next(it)
        scale_hbm = next(it)
        if with_residual:
            w_hbm = next(it)
            d_hbm = next(it)
        zero_hbm = next(it)
        p_out = next(it)
        if with_residual:
            r_out = next(it)
        if with_gather:
            zloc = next(it)
        acc = next(it)
        srcb = next(it)
        dstb = next(it)
        scaleb = next(it)
        mb = next(it)
        nmb = next(it)
        if with_gather:
            zsb = next(it)
            zdb = next(it)
        else:
            db = next(it)
        if with_residual:
            wb = next(it)
            db = next(it)
            rb = next(it)
        sem = next(it)
        sem_g = next(it)
        sem_s = next(it)

        c = lax.axis_index("c")
        s = lax.axis_index("s")

        for j in range(-(-_NRC // _NS)):
            cid = s + _NS * j

            @pl.when(cid < _NRC)
            def _():
                r0 = cid * _RC
                if with_gather:
                    pltpu.sync_copy(
                        main_hbm.at[pl.ds(r0, _RC), :],
                        zloc.at[pl.ds(r0, _RC), :],
                    )
                pltpu.sync_copy(
                    zero_hbm.at[pl.ds(r0, _RC), :],
                    acc.at[pl.ds(r0, _RC), :],
                )

        plsc.subcore_barrier()

        base = (c * _NS + s) * _EW
        lanes = lax.iota(jnp.int32, 16)
        s_row = lanes >> 3
        s_col = lanes & 7

        def chunk_body(t, carry):
            off = base + t * _CH
            cps = [
                pltpu.async_copy(src_hbm.at[pl.ds(off, _CH)], srcb, sem),
                pltpu.async_copy(dst_hbm.at[pl.ds(off, _CH)], dstb, sem),
                pltpu.async_copy(scale_hbm.at[pl.ds(off, _CH)], scaleb, sem),
            ]
            if with_residual:
                cps.append(
                    pltpu.async_copy(w_hbm.at[pl.ds(off, _CH)], wb, sem)
                )
                cps.append(
                    pltpu.async_copy(d_hbm.at[pl.ds(off, _CH), :], db, sem)
                )
            if not with_gather:
                cps.append(
                    pltpu.async_copy(main_hbm.at[pl.ds(off, _CH), :], db, sem)
                )
            for cp in cps:
                cp.wait()
            if with_gather:
                g1 = pltpu.async_copy(zloc.at[srcb], zsb, sem_g)
                g2 = pltpu.async_copy(zloc.at[dstb], zdb, sem_g)
                g1.wait()
                g2.wait()

            def vbody(v, carry2):
                rows = v * 2 + s_row
                a = plsc.load_gather(scaleb, [rows])
                if with_gather:
                    zs = plsc.load_gather(zsb, [rows, s_col])
                    zd = plsc.load_gather(zdb, [rows, s_col])
                    diff = zs - zd
                    m = a * diff
                    if with_residual:
                        wv = plsc.load_gather(wb, [rows])
                        dv = plsc.load_gather(db, [rows, s_col])
                        plsc.store_scatter(rb, [rows, s_col], wv * diff - dv)
                else:
                    dv = plsc.load_gather(db, [rows, s_col])
                    m = a * dv
                plsc.store_scatter(mb, [rows, s_col], m)
                plsc.store_scatter(nmb, [rows, s_col], -m)
                return carry2

            lax.fori_loop(0, _VPC, vbody, 0, unroll=8)

            s1 = pltpu.async_copy(mb, acc.at[srcb], sem_s, add=True)
            s2 = pltpu.async_copy(nmb, acc.at[dstb], sem_s, add=True)
            s1.wait()
            s2.wait()
            if with_residual:
                pltpu.async_copy(rb, r_out.at[pl.ds(off, _CH), :], sem).wait()
            return carry

        lax.fori_loop(0, _NCHUNK, chunk_body, 0)
        plsc.subcore_barrier()

        for j in range(-(-_NRC // _NS)):
            cid = s + _NS * j

            @pl.when(cid < _NRC)
            def _():
                r0 = cid * _RC
                pltpu.sync_copy(
                    acc.at[pl.ds(r0, _RC), :],
                    p_out.at[c, pl.ds(r0, _RC), :],
                )

    return pl.kernel(
        body,
        out_type=out_type,
        mesh=mesh,
        scratch_types=_edge_scratch(with_gather, with_residual),
        compiler_params=pltpu.CompilerParams(
            needs_layout_passes=False, use_tc_tiling_on_sc=False
        ),
    )


def _merge(z, p, pb):
    zr = z.reshape(_N * _C // 128, 128)
    pr = p.reshape(_NC, _N * _C // 128, 128)
    pbr = pb.reshape(_NC, _N * _C // 128, 128)

    def body(z_ref, p_ref, pb_ref, o_ref):
        o_ref[...] = (
            z_ref[...] - p_ref[0] - p_ref[1] + pb_ref[0] + pb_ref[1]
        )

    out = pl.pallas_call(
        body,
        out_shape=jax.ShapeDtypeStruct((_N * _C // 128, 128), jnp.float32),
    )(zr, pr, pbr)
    return out.reshape(_N, _C)


def kernel(D, edge_index, edge_weights):
    src = edge_index[0]
    dst = edge_index[1]
    w = edge_weights
    aw = _LR * w
    aw2 = aw * w

    b_pass = _make_edge_kernel(with_gather=False, with_residual=False)
    l_pass = _make_edge_kernel(with_gather=True, with_residual=False)
    lr_pass = _make_edge_kernel(with_gather=True, with_residual=True)

    zeros_n = jnp.zeros((_N, _C), jnp.float32)
    zeros_p = jnp.zeros((_NC, _N, _C), jnp.float32)
    [pb] = b_pass(D, src, dst, aw, zeros_n)
    z = _merge(zeros_n, zeros_p, pb)
    for _ in range(8):
        [p] = l_pass(z, src, dst, aw2, zeros_n)
        z = _merge(z, p, pb)
    p, r = lr_pass(z, src, dst, aw2, w, D, zeros_n)
    z = _merge(z, p, pb)
    return z, zeros_n, r


# speedup vs baseline: 39.3348x; 1.0753x over previous
import functools

import jax
import jax.numpy as jnp
from jax import lax
from jax.experimental import pallas as pl
from jax.experimental.pallas import tpu as pltpu
from jax.experimental.pallas import tpu_sc as plsc

_N = 100000
_E = 3200000
_C = 8
_LR = 1e-3
_NC = 2
_NS = 16
_NW = _NC * _NS
_EW = _E // _NW
_RC = 1000
_NRC = _N // _RC


def _edge_scratch(with_gather, with_residual, ch):
    scr = []
    if with_gather:
        scr.append(pltpu.VMEM_SHARED((_N, _C), jnp.float32))
    scr += [
        pltpu.VMEM_SHARED((_N, _C), jnp.float32),
        pltpu.VMEM((ch,), jnp.int32),
        pltpu.VMEM((ch,), jnp.int32),
        pltpu.VMEM((ch,), jnp.float32),
        pltpu.VMEM((ch, _C), jnp.float32),
        pltpu.VMEM((ch, _C), jnp.float32),
    ]
    if with_gather:
        scr += [
            pltpu.VMEM((ch, _C), jnp.float32),
            pltpu.VMEM((ch, _C), jnp.float32),
        ]
    else:
        scr.append(pltpu.VMEM((ch, _C), jnp.float32))
    if with_residual:
        scr += [
            pltpu.VMEM((ch,), jnp.float32),
            pltpu.VMEM((ch, _C), jnp.float32),
            pltpu.VMEM((ch, _C), jnp.float32),
        ]
    scr.append(pltpu.SemaphoreType.DMA)
    scr.append(pltpu.SemaphoreType.DMA)
    scr.append(pltpu.SemaphoreType.DMA)
    return scr


def _make_edge_kernel(with_gather, with_residual, ch):
    mesh = plsc.VectorSubcoreMesh(core_axis_name="c", subcore_axis_name="s")
    out_type = [jax.ShapeDtypeStruct((_NC, _N, _C), jnp.float32)]
    if with_residual:
        out_type.append(jax.ShapeDtypeStruct((_E, _C), jnp.float32))

    def body(*refs):
        it = iter(refs)
        main_hbm = next(it)
        src_hbm = next(it)
        dst_hbm = next(it)
        scale_hbm = next(it)
        if with_residual:
            w_hbm = next(it)
            d_hbm = next(it)
        zero_hbm = next(it)
        p_out = next(it)
        if with_residual:
            r_out = next(it)
        if with_gather:
            zloc = next(it)
        acc = next(it)
        srcb = next(it)
        dstb = next(it)
        scaleb = next(it)
        mb = next(it)
        nmb = next(it)
        if with_gather:
            zsb = next(it)
            zdb = next(it)
        else:
            db = next(it)
        if with_residual:
            wb = next(it)
            db = next(it)
            rb = next(it)
        sem = next(it)
        sem_g = next(it)
        sem_s = next(it)

        c = lax.axis_index("c")
        s = lax.axis_index("s")

        for j in range(-(-_NRC // _NS)):
            cid = s + _NS * j

            @pl.when(cid < _NRC)
            def _():
                r0 = cid * _RC
                if with_gather:
                    pltpu.sync_copy(
                        main_hbm.at[pl.ds(r0, _RC), :],
                        zloc.at[pl.ds(r0, _RC), :],
                    )
                pltpu.sync_copy(
                    zero_hbm.at[pl.ds(r0, _RC), :],
                    acc.at[pl.ds(r0, _RC), :],
                )

        plsc.subcore_barrier()

        base = (c * _NS + s) * _EW
        lanes = lax.iota(jnp.int32, 16)
        s_row = lanes >> 3
        s_col = lanes & 7

        def chunk_body(t, carry):
            off = base + t * ch
            cps = [
                pltpu.async_copy(src_hbm.at[pl.ds(off, ch)], srcb, sem),
                pltpu.async_copy(dst_hbm.at[pl.ds(off, ch)], dstb, sem),
                pltpu.async_copy(scale_hbm.at[pl.ds(off, ch)], scaleb, sem),
            ]
            if with_residual:
                cps.append(
                    pltpu.async_copy(w_hbm.at[pl.ds(off, ch)], wb, sem)
                )
                cps.append(
                    pltpu.async_copy(d_hbm.at[pl.ds(off, ch), :], db, sem)
                )
            if not with_gather:
                cps.append(
                    pltpu.async_copy(main_hbm.at[pl.ds(off, ch), :], db, sem)
                )
            for cp in cps:
                cp.wait()
            if with_gather:
                g1 = pltpu.async_copy(zloc.at[srcb], zsb, sem_g)
                g2 = pltpu.async_copy(zloc.at[dstb], zdb, sem_g)
                g1.wait()
                g2.wait()

            def vbody(v, carry2):
                rows = v * 2 + s_row
                a = plsc.load_gather(scaleb, [rows])
                if with_gather:
                    zs = plsc.load_gather(zsb, [rows, s_col])
                    zd = plsc.load_gather(zdb, [rows, s_col])
                    diff = zs - zd
                    m = a * diff
                    if with_residual:
                        wv = plsc.load_gather(wb, [rows])
                        dv = plsc.load_gather(db, [rows, s_col])
                        plsc.store_scatter(rb, [rows, s_col], wv * diff - dv)
                else:
                    dv = plsc.load_gather(db, [rows, s_col])
                    m = a * dv
                plsc.store_scatter(mb, [rows, s_col], m)
                plsc.store_scatter(nmb, [rows, s_col], -m)
                return carry2

            lax.fori_loop(0, (ch * _C // 16), vbody, 0, unroll=8)

            s1 = pltpu.async_copy(mb, acc.at[srcb], sem_s, add=True)
            s2 = pltpu.async_copy(nmb, acc.at[dstb], sem_s, add=True)
            s1.wait()
            s2.wait()
            if with_residual:
                pltpu.async_copy(rb, r_out.at[pl.ds(off, ch), :], sem).wait()
            return carry

        lax.fori_loop(0, (_EW // ch), chunk_body, 0)
        plsc.subcore_barrier()

        for j in range(-(-_NRC // _NS)):
            cid = s + _NS * j

            @pl.when(cid < _NRC)
            def _():
                r0 = cid * _RC
                pltpu.sync_copy(
                    acc.at[pl.ds(r0, _RC), :],
                    p_out.at[c, pl.ds(r0, _RC), :],
                )

    return pl.kernel(
        body,
        out_type=out_type,
        mesh=mesh,
        scratch_types=_edge_scratch(with_gather, with_residual, ch),
        compiler_params=pltpu.CompilerParams(
            needs_layout_passes=False, use_tc_tiling_on_sc=False
        ),
    )


def _merge(z, p, pb):
    zr = z.reshape(_N * _C // 128, 128)
    pr = p.reshape(_NC, _N * _C // 128, 128)
    pbr = pb.reshape(_NC, _N * _C // 128, 128)

    def body(z_ref, p_ref, pb_ref, o_ref):
        o_ref[...] = (
            z_ref[...] - p_ref[0] - p_ref[1] + pb_ref[0] + pb_ref[1]
        )

    out = pl.pallas_call(
        body,
        out_shape=jax.ShapeDtypeStruct((_N * _C // 128, 128), jnp.float32),
    )(zr, pr, pbr)
    return out.reshape(_N, _C)


def kernel(D, edge_index, edge_weights):
    src = edge_index[0]
    dst = edge_index[1]
    w = edge_weights
    aw = _LR * w
    aw2 = aw * w

    b_pass = _make_edge_kernel(with_gather=False, with_residual=False, ch=800)
    l_pass = _make_edge_kernel(with_gather=True, with_residual=False, ch=800)
    lr_pass = _make_edge_kernel(with_gather=True, with_residual=True, ch=400)

    zeros_n = jnp.zeros((_N, _C), jnp.float32)
    zeros_p = jnp.zeros((_NC, _N, _C), jnp.float32)
    [pb] = b_pass(D, src, dst, aw, zeros_n)
    z = _merge(zeros_n, zeros_p, pb)
    for _ in range(8):
        [p] = l_pass(z, src, dst, aw2, zeros_n)
        z = _merge(z, p, pb)
    p, r = lr_pass(z, src, dst, aw2, w, D, zeros_n)
    z = _merge(z, p, pb)
    return z, zeros_n, r


# speedup vs baseline: 40.5487x; 1.0309x over previous
import functools

import jax
import jax.numpy as jnp
from jax import lax
from jax.experimental import pallas as pl
from jax.experimental.pallas import tpu as pltpu
from jax.experimental.pallas import tpu_sc as plsc

_N = 100000
_E = 3200000
_C = 8
_LR = 1e-3
_NC = 2
_NS = 16
_NW = _NC * _NS
_EW = _E // _NW
_RC = 1000
_NRC = _N // _RC


def _edge_scratch(with_gather, with_residual, ch):
    scr = []
    if with_gather:
        scr.append(pltpu.VMEM_SHARED((_N, _C), jnp.float32))
    scr += [
        pltpu.VMEM_SHARED((_N, _C), jnp.float32),
        pltpu.VMEM((ch,), jnp.int32),
        pltpu.VMEM((ch,), jnp.int32),
        pltpu.VMEM((ch,), jnp.float32),
        pltpu.VMEM((ch, _C), jnp.float32),
        pltpu.VMEM((ch, _C), jnp.float32),
    ]
    if with_gather:
        scr += [
            pltpu.VMEM((ch, _C), jnp.float32),
            pltpu.VMEM((ch, _C), jnp.float32),
        ]
    else:
        scr.append(pltpu.VMEM((ch, _C), jnp.float32))
    if with_residual:
        scr += [
            pltpu.VMEM((ch,), jnp.float32),
            pltpu.VMEM((ch, _C), jnp.float32),
            pltpu.VMEM((ch, _C), jnp.float32),
        ]
    for _ in range(5):
        scr.append(pltpu.SemaphoreType.DMA)
    return scr


def _make_edge_kernel(with_gather, with_residual, ch):
    mesh = plsc.VectorSubcoreMesh(core_axis_name="c", subcore_axis_name="s")
    out_type = [jax.ShapeDtypeStruct((_NC, _N, _C), jnp.float32)]
    if with_residual:
        out_type.append(jax.ShapeDtypeStruct((_E, _C), jnp.float32))

    def body(*refs):
        it = iter(refs)
        main_hbm = next(it)
        src_hbm = next(it)
        dst_hbm = next(it)
        scale_hbm = next(it)
        if with_residual:
            w_hbm = next(it)
            d_hbm = next(it)
        zero_hbm = next(it)
        p_out = next(it)
        if with_residual:
            r_out = next(it)
        if with_gather:
            zloc = next(it)
        acc = next(it)
        srcb = next(it)
        dstb = next(it)
        scaleb = next(it)
        mb = next(it)
        nmb = next(it)
        if with_gather:
            zsb = next(it)
            zdb = next(it)
        else:
            db = next(it)
        if with_residual:
            wb = next(it)
            db = next(it)
            rb = next(it)
        sem = next(it)
        sem_ga = next(it)
        sem_gb = next(it)
        sem_sa = next(it)
        sem_sb = next(it)

        c = lax.axis_index("c")
        s = lax.axis_index("s")

        for j in range(-(-_NRC // _NS)):
            cid = s + _NS * j

            @pl.when(cid < _NRC)
            def _():
                r0 = cid * _RC
                if with_gather:
                    pltpu.sync_copy(
                        main_hbm.at[pl.ds(r0, _RC), :],
                        zloc.at[pl.ds(r0, _RC), :],
                    )
                pltpu.sync_copy(
                    zero_hbm.at[pl.ds(r0, _RC), :],
                    acc.at[pl.ds(r0, _RC), :],
                )

        plsc.subcore_barrier()

        base = (c * _NS + s) * _EW
        lanes = lax.iota(jnp.int32, 16)
        s_row = lanes >> 3
        s_col = lanes & 7

        ch2 = ch // 2
        nv2 = ch * _C // 32

        def vrange(v0, carry0):
            def vbody(v, carry2):
                rows = v * 2 + s_row
                a = plsc.load_gather(scaleb, [rows])
                if with_gather:
                    zs = plsc.load_gather(zsb, [rows, s_col])
                    zd = plsc.load_gather(zdb, [rows, s_col])
                    diff = zs - zd
                    m = a * diff
                    if with_residual:
                        wv = plsc.load_gather(wb, [rows])
                        dv = plsc.load_gather(db, [rows, s_col])
                        plsc.store_scatter(rb, [rows, s_col], wv * diff - dv)
                else:
                    dv = plsc.load_gather(db, [rows, s_col])
                    m = a * dv
                plsc.store_scatter(mb, [rows, s_col], m)
                plsc.store_scatter(nmb, [rows, s_col], -m)
                return carry2

            lax.fori_loop(v0, v0 + nv2, vbody, carry0, unroll=8)

        def chunk_body(t, carry):
            off = base + t * ch
            cps = [
                pltpu.async_copy(src_hbm.at[pl.ds(off, ch)], srcb, sem),
                pltpu.async_copy(dst_hbm.at[pl.ds(off, ch)], dstb, sem),
                pltpu.async_copy(scale_hbm.at[pl.ds(off, ch)], scaleb, sem),
            ]
            if with_residual:
                cps.append(
                    pltpu.async_copy(w_hbm.at[pl.ds(off, ch)], wb, sem)
                )
                cps.append(
                    pltpu.async_copy(d_hbm.at[pl.ds(off, ch), :], db, sem)
                )
            if not with_gather:
                cps.append(
                    pltpu.async_copy(main_hbm.at[pl.ds(off, ch), :], db, sem)
                )
            for cp in cps:
                cp.wait()
            if with_gather:
                ga = [
                    pltpu.async_copy(
                        zloc.at[srcb.at[pl.ds(0, ch2)]],
                        zsb.at[pl.ds(0, ch2), :], sem_ga),
                    pltpu.async_copy(
                        zloc.at[dstb.at[pl.ds(0, ch2)]],
                        zdb.at[pl.ds(0, ch2), :], sem_ga),
                ]
                gb = [
                    pltpu.async_copy(
                        zloc.at[srcb.at[pl.ds(ch2, ch2)]],
                        zsb.at[pl.ds(ch2, ch2), :], sem_gb),
                    pltpu.async_copy(
                        zloc.at[dstb.at[pl.ds(ch2, ch2)]],
                        zdb.at[pl.ds(ch2, ch2), :], sem_gb),
                ]
                for cp in ga:
                    cp.wait()
                vrange(0, 0)
                sa = [
                    pltpu.async_copy(
                        mb.at[pl.ds(0, ch2), :],
                        acc.at[srcb.at[pl.ds(0, ch2)]], sem_sa, add=True),
                    pltpu.async_copy(
                        nmb.at[pl.ds(0, ch2), :],
                        acc.at[dstb.at[pl.ds(0, ch2)]], sem_sa, add=True),
                ]
                for cp in gb:
                    cp.wait()
                vrange(nv2, 0)
                sb = [
                    pltpu.async_copy(
                        mb.at[pl.ds(ch2, ch2), :],
                        acc.at[srcb.at[pl.ds(ch2, ch2)]], sem_sb, add=True),
                    pltpu.async_copy(
                        nmb.at[pl.ds(ch2, ch2), :],
                        acc.at[dstb.at[pl.ds(ch2, ch2)]], sem_sb, add=True),
                ]
                for cp in sa + sb:
                    cp.wait()
            else:
                vrange(0, 0)
                sa = [
                    pltpu.async_copy(
                        mb.at[pl.ds(0, ch2), :],
                        acc.at[srcb.at[pl.ds(0, ch2)]], sem_sa, add=True),
                    pltpu.async_copy(
                        nmb.at[pl.ds(0, ch2), :],
                        acc.at[dstb.at[pl.ds(0, ch2)]], sem_sa, add=True),
                ]
                vrange(nv2, 0)
                sb = [
                    pltpu.async_copy(
                        mb.at[pl.ds(ch2, ch2), :],
                        acc.at[srcb.at[pl.ds(ch2, ch2)]], sem_sb, add=True),
                    pltpu.async_copy(
                        nmb.at[pl.ds(ch2, ch2), :],
                        acc.at[dstb.at[pl.ds(ch2, ch2)]], sem_sb, add=True),
                ]
                for cp in sa + sb:
                    cp.wait()
            if with_residual:
                pltpu.async_copy(rb, r_out.at[pl.ds(off, ch), :], sem).wait()
            return carry

        lax.fori_loop(0, (_EW // ch), chunk_body, 0)
        plsc.subcore_barrier()

        for j in range(-(-_NRC // _NS)):
            cid = s + _NS * j

            @pl.when(cid < _NRC)
            def _():
                r0 = cid * _RC
                pltpu.sync_copy(
                    acc.at[pl.ds(r0, _RC), :],
                    p_out.at[c, pl.ds(r0, _RC), :],
                )

    return pl.kernel(
        body,
        out_type=out_type,
        mesh=mesh,
        scratch_types=_edge_scratch(with_gather, with_residual, ch),
        compiler_params=pltpu.CompilerParams(
            needs_layout_passes=False, use_tc_tiling_on_sc=False
        ),
    )


def _merge(z, p, pb):
    zr = z.reshape(_N * _C // 128, 128)
    pr = p.reshape(_NC, _N * _C // 128, 128)
    pbr = pb.reshape(_NC, _N * _C // 128, 128)

    def body(z_ref, p_ref, pb_ref, o_ref):
        o_ref[...] = (
            z_ref[...] - p_ref[0] - p_ref[1] + pb_ref[0] + pb_ref[1]
        )

    out = pl.pallas_call(
        body,
        out_shape=jax.ShapeDtypeStruct((_N * _C // 128, 128), jnp.float32),
    )(zr, pr, pbr)
    return out.reshape(_N, _C)


def kernel(D, edge_index, edge_weights):
    src = edge_index[0]
    dst = edge_index[1]
    w = edge_weights
    aw = _LR * w
    aw2 = aw * w

    b_pass = _make_edge_kernel(with_gather=False, with_residual=False, ch=800)
    l_pass = _make_edge_kernel(with_gather=True, with_residual=False, ch=800)
    lr_pass = _make_edge_kernel(with_gather=True, with_residual=True, ch=400)

    zeros_n = jnp.zeros((_N, _C), jnp.float32)
    zeros_p = jnp.zeros((_NC, _N, _C), jnp.float32)
    [pb] = b_pass(D, src, dst, aw, zeros_n)
    z = _merge(zeros_n, zeros_p, pb)
    for _ in range(8):
        [p] = l_pass(z, src, dst, aw2, zeros_n)
        z = _merge(z, p, pb)
    p, r = lr_pass(z, src, dst, aw2, w, D, zeros_n)
    z = _merge(z, p, pb)
    return z, zeros_n, r
